# baseline (speedup 1.0000x reference)
import functools
import math

import jax
import jax.numpy as jnp
from jax import lax
from jax.experimental import pallas as pl
from jax.experimental.pallas import tpu as pltpu
from jax.experimental.pallas import tpu_sc as plsc

D_MODEL = 64
SCALE = math.sqrt(D_MODEL)
NC, NS, L = 2, 16, 16
NW = NC * NS
BLK = 128
NBUF = 2


@functools.cache
def _make_emb(S0, S1, V):
    assert S0 == NW * BLK and D_MODEL % 8 == 0
    dtiles = D_MODEL // 8
    rows_per_s1 = dtiles * (S0 // BLK) * 8 // 8
    out_rows = S1 * D_MODEL * S0 // BLK // 8 * 8

    mesh = plsc.VectorSubcoreMesh(core_axis_name="c", subcore_axis_name="s")

    @functools.partial(
        pl.kernel,
        out_type=jax.ShapeDtypeStruct((S1 * (D_MODEL // 8) * (S0 // BLK) * 8, BLK), jnp.float32),
        mesh=mesh,
        scratch_types=[
            pltpu.VMEM((S1, BLK), jnp.int32),
            pltpu.VMEM((NBUF, BLK, D_MODEL), jnp.float32),
            pltpu.VMEM((NBUF, D_MODEL, BLK), jnp.float32),
            pltpu.SemaphoreType.DMA,
            pltpu.SemaphoreType.DMA,
            pltpu.SemaphoreType.DMA,
            pltpu.SemaphoreType.DMA,
        ],
        compiler_params=pltpu.CompilerParams(
            use_tc_tiling_on_sc=False, needs_layout_passes=False
        ),
    )
    def emb(lut_hbm, idx_hbm, out_hbm, idx_v, buf, bufT, g0, g1, w0, w1):
        gsems = (g0, g1)
        wsems = (w0, w1)
        wid = lax.axis_index("s") * NC + lax.axis_index("c")
        pltpu.sync_copy(idx_hbm.at[:, pl.ds(wid * BLK, BLK)], idx_v)

        def gather_refs(ch, b):
            src = lut_hbm.at[idx_v.at[ch]]
            dst = buf.at[b]
            return src, dst

        def write_refs(ch, b, dt):
            src = bufT.at[b].at[pl.ds(dt * 8, 8)]
            dst = out_hbm.at[pl.ds(ch * (D_MODEL * (S0 // BLK)) + dt * (8 * (S0 // BLK)) + wid * 8, 8)]
            return src, dst

        pltpu.async_copy(*gather_refs(0, 0), gsems[0])

        @pl.loop(0, S1, step=NBUF)
        def outer(i):
            for b in range(NBUF):
                ch = i + b

                @pl.when(ch + 1 < S1)
                def _():
                    src, dst = gather_refs(ch + 1, (b + 1) % NBUF)
                    pltpu.async_copy(src, dst, gsems[(b + 1) % NBUF])

                @pl.when(ch >= NBUF)
                def _():
                    for dt in range(dtiles):
                        src, dst = write_refs(ch - NBUF, b, dt)
                        pltpu.make_async_copy(src, dst, wsems[b]).wait()

                src, dst = gather_refs(ch, b)
                pltpu.make_async_copy(src, dst, gsems[b]).wait()

                svs = []
                for g in range(BLK // L):
                    iv = idx_v[ch, pl.ds(g * L, L)]
                    svs.append(
                        jnp.where(iv == 0, jnp.float32(0.0), jnp.float32(SCALE))
                    )

                @pl.loop(0, D_MODEL, unroll=4)
                def trans_d(d):
                    cols = jnp.full((L,), d, jnp.int32)
                    for g in range(BLK // L):
                        rows = g * L + lax.iota(jnp.int32, L)
                        v = plsc.load_gather(buf.at[b], [rows, cols])
                        bufT[b, d, pl.ds(g * L, L)] = v * svs[g]

                for dt in range(dtiles):
                    src, dst = write_refs(ch, b, dt)
                    pltpu.async_copy(src, dst, wsems[b])

        for b in range(NBUF):
            ch = S1 - NBUF + b
            for dt in range(dtiles):
                src, dst = write_refs(ch, b, dt)
                pltpu.make_async_copy(src, dst, wsems[b]).wait()

    return emb


def kernel(x, lut):
    s0, s1 = x.shape
    xt = x.T.astype(jnp.int32)
    outp = _make_emb(s0, s1, lut.shape[0])(lut, xt)
    out5 = outp.reshape(s1, D_MODEL // 8, s0 // BLK, 8, BLK)
    return out5.transpose(2, 4, 0, 1, 3).reshape(s0, s1, D_MODEL)
